# baseline (speedup 1.0000x reference)
import functools

import jax
import jax.numpy as jnp
from jax import lax
from jax.experimental import pallas as pl
from jax.experimental.pallas import tpu as pltpu
from jax.experimental.pallas import tpu_sc as plsc

B = 16384
V = 1000000
D = 16
EPS = 1e-3
NC = 2
NS = 16
NW = NC * NS
BPW = B // NW
NK = BPW // 128
NF = 13


def _rsqrt16(x):
    i = lax.bitcast_convert_type(x, jnp.int32)
    i = jnp.int32(0x5F3759DF) - lax.shift_right_arithmetic(i, 1)
    y = lax.bitcast_convert_type(i, jnp.float32)
    for _ in range(3):
        y = y * (jnp.float32(1.5) - jnp.float32(0.5) * x * y * y)
    return y


def _sc_body(table, idxs, vals, params, out, idx_v, rows_v, out_v, vals_v,
             params_v, gsem, osem):
    wid = lax.axis_index("s") * NC + lax.axis_index("c")

    pltpu.sync_copy(params, params_v)
    pltpu.sync_copy(idxs.at[:, wid], idx_v)
    pltpu.sync_copy(vals.at[:, wid], vals_v)

    bn_scale = params_v[0, :]
    bn_shift = params_v[1, :]

    def fire(f, slot):
        return [
            pltpu.async_copy(
                table.at[idx_v.at[f, j]],
                rows_v.at[slot, pl.ds(j * 128, 128), 0],
                gsem,
            )
            for j in range(NK)
        ]

    pending = {0: fire(0, 0)}
    out_pending = {}
    for f in range(NF):
        slot = f % 2
        if f + 1 < NF:
            if f - 1 in out_pending:
                out_pending.pop(f - 1).wait()
            pending[f + 1] = fire(f + 1, (f + 1) % 2)
        for h in pending.pop(f):
            h.wait()

        rv = rows_v.at[slot, :, 0]
        ov = out_v.at[slot]
        if f < 10:
            def bn_body(ib, _, rv=rv, ov=ov):
                for u in range(8):
                    row = rv[ib * 8 + u] * bn_scale + bn_shift
                    ov[ib, pl.ds(u * 16, 16)] = row
                return 0
            lax.fori_loop(0, BPW // 8, bn_body, 0)
        else:
            k = f - 10
            g = params_v[2 + k, :]
            bt = params_v[5 + k, :]

            def ln_body(ib, _, rv=rv, ov=ov, k=k, g=g, bt=bt):
                cvec = vals_v[k, pl.ds(ib * 16, 16)]
                for u in range(16):
                    i = ib * 16 + u
                    row = rv[i] * cvec[u]
                    mu = jnp.sum(row) * jnp.float32(1.0 / D)
                    ctr = row - mu
                    var = jnp.sum(ctr * ctr) * jnp.float32(1.0 / D)
                    r = _rsqrt16(jnp.full((D,), var + jnp.float32(EPS),
                                          jnp.float32))
                    ov[ib * 2 + u // 8, pl.ds((u % 8) * 16, 16)] = (
                        ctr * r * g + bt)
                return 0
            lax.fori_loop(0, BPW // 16, ln_body, 0)

        out_pending[f] = pltpu.async_copy(
            ov, out.at[f, pl.ds(wid * (BPW * D // 128), BPW * D // 128)],
            osem)

    for f in sorted(out_pending):
        out_pending.pop(f).wait()


def _eye16():
    r = jax.lax.broadcasted_iota(jnp.int32, (16, 16), 0)
    c = jax.lax.broadcasted_iota(jnp.int32, (16, 16), 1)
    return jnp.where(r == c, jnp.float32(1.0), jnp.float32(0.0))


def _tc_transpose_table(tt):
    CB = 8192

    def body(x_ref, o_ref):
        eye = _eye16()
        o_ref[...] = jax.lax.dot_general(
            x_ref[...], eye, (((0,), (0,)), ((), ())),
            precision=jax.lax.Precision.HIGHEST,
            preferred_element_type=jnp.float32)

    return pl.pallas_call(
        body,
        grid=(pl.cdiv(V, CB),),
        in_specs=[pl.BlockSpec((16, CB), lambda i: (0, i))],
        out_specs=pl.BlockSpec((CB, 16), lambda i: (i, 0)),
        out_shape=jax.ShapeDtypeStruct((V, D), jnp.float32),
    )(tt)


def _tc_transpose_out(o2):
    CB = 16384

    def body(x_ref, o_ref):
        eye = _eye16()
        o_ref[0] = jax.lax.dot_general(
            eye, x_ref[0], (((1,), (1,)), ((), ())),
            precision=jax.lax.Precision.HIGHEST,
            preferred_element_type=jnp.float32)

    return pl.pallas_call(
        body,
        grid=(NF, B // CB),
        in_specs=[pl.BlockSpec((1, CB, 16), lambda f, i: (f, i, 0))],
        out_specs=pl.BlockSpec((1, 16, CB), lambda f, i: (f, 0, i)),
        out_shape=jax.ShapeDtypeStruct((NF, D, B), jnp.float32),
    )(o2)


@jax.jit
def _sc_call(table, idxs, vals, params):
    mesh = plsc.VectorSubcoreMesh(
        core_axis_name="c", subcore_axis_name="s",
        num_cores=NC, num_subcores=NS)
    return pl.kernel(
        _sc_body,
        out_type=jax.ShapeDtypeStruct((NF, B * D // 128, 128), jnp.float32),
        mesh=mesh,
        scratch_types=[
            pltpu.VMEM((NF, NK, 128), jnp.int32),
            pltpu.VMEM((2, BPW, 1, D), jnp.float32),
            pltpu.VMEM((2, BPW * D // 128, 128), jnp.float32),
            pltpu.VMEM((3, BPW), jnp.float32),
            pltpu.VMEM((8, D), jnp.float32),
            pltpu.SemaphoreType.DMA,
            pltpu.SemaphoreType.DMA,
        ],
        compiler_params=pltpu.CompilerParams(
            needs_layout_passes=False, use_tc_tiling_on_sc=False),
    )(table, idxs, vals, params)


def kernel(uid, iid, utag1, utag2, utag3, utag4, itag1, itag2, itag3, itag4,
           itag4_origin_key, itag4_square_key, itag4_cube_key,
           itag4_origin_value, itag4_square_value, itag4_cube_value,
           table, bn_gamma, bn_beta, bn_mean, bn_var, ln_gamma, ln_beta):
    idxs = jnp.stack([uid, iid, utag1, utag2, utag3, utag4,
                      itag1, itag2, itag3, itag4,
                      itag4_origin_key, itag4_square_key, itag4_cube_key],
                     axis=0).reshape(NF, NW, NK, 128)
    vals = jnp.stack([itag4_origin_value, itag4_square_value,
                      itag4_cube_value], axis=0).reshape(3, NW, BPW)
    bn_scale = bn_gamma / jnp.sqrt(bn_var + EPS)
    bn_shift = bn_beta - bn_mean * bn_scale
    params = jnp.concatenate(
        [bn_scale[None, :], bn_shift[None, :], ln_gamma, ln_beta], axis=0)
    table_lin = _tc_transpose_table(jnp.transpose(table))
    out = _sc_call(table_lin, idxs, vals, params)
    o3 = _tc_transpose_out(jnp.reshape(out, (NF, B, D)))
    return jnp.transpose(o3, (2, 0, 1))


# speedup vs baseline: 4.5415x; 4.5415x over previous
import jax
import jax.numpy as jnp
from jax import lax
from jax.experimental import pallas as pl
from jax.experimental.pallas import tpu as pltpu
from jax.experimental.pallas import tpu_sc as plsc

B = 16384
V = 1000000
D = 16
EPS = 1e-3
NC = 2
NS = 16
NW = NC * NS
BPW = B // NW
NK = BPW // 128
NF = 13


def _rsqrt16(x):
    i = lax.bitcast_convert_type(x, jnp.int32)
    i = jnp.int32(0x5F3759DF) - lax.shift_right_arithmetic(i, 1)
    y = lax.bitcast_convert_type(i, jnp.float32)
    for _ in range(3):
        y = y * (jnp.float32(1.5) - jnp.float32(0.5) * x * y * y)
    return y


def _sc_body(*refs):
    chans = refs[:D]
    idxs, vals, params, out = refs[D:D + 4]
    idx_v, tr_v, vals_v, params_v, gsem, osem = refs[D + 4:]

    wid = lax.axis_index("s") * NC + lax.axis_index("c")

    pltpu.sync_copy(params, params_v)
    pltpu.sync_copy(idxs.at[:, wid], idx_v)
    pltpu.sync_copy(vals.at[:, wid], vals_v)

    def fire(i, carry):
        f = i >> 2
        j = i & 3
        for c in range(D):
            pltpu.async_copy(
                chans[c].at[idx_v.at[f, j]], tr_v.at[f, j, c], gsem)
        return carry
    lax.fori_loop(0, NF * NK, fire, 0)

    pltpu.make_async_copy(
        out.at[0, 0, pl.ds(0, NF), :, pl.ds(0, NK * 128 // 8)]
        if False else out.at[pl.ds(0, NF), 0, pl.ds(0, NK), :, :],
        tr_v, gsem).wait()

    bn_scale = params_v[0, :]
    bn_shift = params_v[1, :]

    out_handles = []
    for f in range(NF):
        if f < 10:
            def bn_body(g, carry, f=f):
                j = g >> 3
                off = (g & 7) * 16
                for c in range(D):
                    sc = bn_scale[c]
                    sh = bn_shift[c]
                    t = tr_v[f, j, c, pl.ds(off, 16)]
                    tr_v[f, j, c, pl.ds(off, 16)] = t * sc + sh
                return carry
            lax.fori_loop(0, BPW // 16, bn_body, 0)
        else:
            k = f - 10
            gvec = params_v[2 + k, :]
            bvec = params_v[5 + k, :]

            def ln_body(g, carry, f=f, k=k, gvec=gvec, bvec=bvec):
                j = g >> 3
                off = (g & 7) * 16
                cvec = vals_v[k, pl.ds(g * 16, 16)]
                ts = [tr_v[f, j, c, pl.ds(off, 16)] for c in range(D)]
                s = ts[0]
                for c in range(1, D):
                    s = s + ts[c]
                mu = s * jnp.float32(1.0 / D)
                ctr = [t - mu for t in ts]
                v = ctr[0] * ctr[0]
                for c in range(1, D):
                    v = v + ctr[c] * ctr[c]
                var = v * jnp.float32(1.0 / D)
                r = _rsqrt16(cvec * cvec * var + jnp.float32(EPS))
                s2 = cvec * r
                for c in range(D):
                    tr_v[f, j, c, pl.ds(off, 16)] = (
                        ctr[c] * s2 * gvec[c] + bvec[c])
                return carry
            lax.fori_loop(0, BPW // 16, ln_body, 0)

        for g in range(2):
            out_handles.append(pltpu.async_copy(
                tr_v.at[f, :, pl.ds(g * 8, 8), :],
                out.at[f, g, pl.ds(wid * NK, NK), :, :],
                osem))

    for h in out_handles:
        h.wait()


def _tc_split_channels(tt):
    CB = 32768

    def body(x_ref, *o_refs):
        for c in range(D):
            o_refs[c][...] = x_ref[c, :]

    return pl.pallas_call(
        body,
        grid=(pl.cdiv(V, CB),),
        in_specs=[pl.BlockSpec((D, CB), lambda i: (0, i))],
        out_specs=[pl.BlockSpec((CB,), lambda i: (i,))] * D,
        out_shape=[jax.ShapeDtypeStruct((V,), jnp.float32)] * D,
    )(tt)


@jax.jit
def _sc_call(chans, idxs, vals, params):
    mesh = plsc.VectorSubcoreMesh(
        core_axis_name="c", subcore_axis_name="s",
        num_cores=NC, num_subcores=NS)
    return pl.kernel(
        _sc_body,
        out_type=jax.ShapeDtypeStruct((NF, 2, B // 128, 8, 128),
                                      jnp.float32),
        mesh=mesh,
        scratch_types=[
            pltpu.VMEM((NF, NK, 128), jnp.int32),
            pltpu.VMEM((NF, NK, D, 128), jnp.float32),
            pltpu.VMEM((3, BPW), jnp.float32),
            pltpu.VMEM((8, D), jnp.float32),
            pltpu.SemaphoreType.DMA,
            pltpu.SemaphoreType.DMA,
        ],
        compiler_params=pltpu.CompilerParams(
            needs_layout_passes=False, use_tc_tiling_on_sc=False),
    )(*chans, idxs, vals, params)


def kernel(uid, iid, utag1, utag2, utag3, utag4, itag1, itag2, itag3, itag4,
           itag4_origin_key, itag4_square_key, itag4_cube_key,
           itag4_origin_value, itag4_square_value, itag4_cube_value,
           table, bn_gamma, bn_beta, bn_mean, bn_var, ln_gamma, ln_beta):
    idxs = jnp.stack([uid, iid, utag1, utag2, utag3, utag4,
                      itag1, itag2, itag3, itag4,
                      itag4_origin_key, itag4_square_key, itag4_cube_key],
                     axis=0).reshape(NF, NW, NK, 128)
    vals = jnp.stack([itag4_origin_value, itag4_square_value,
                      itag4_cube_value], axis=0).reshape(3, NW, BPW)
    bn_scale = bn_gamma / jnp.sqrt(bn_var + EPS)
    bn_shift = bn_beta - bn_mean * bn_scale
    params = jnp.concatenate(
        [bn_scale[None, :], bn_shift[None, :], ln_gamma, ln_beta], axis=0)
    chans = _tc_split_channels(jnp.transpose(table))
    o = _sc_call(chans, idxs, vals, params)
    return jnp.transpose(o, (2, 4, 0, 1, 3)).reshape(B, NF, D)


# speedup vs baseline: 4.6570x; 1.0254x over previous
import jax
import jax.numpy as jnp
from jax import lax
from jax.experimental import pallas as pl
from jax.experimental.pallas import tpu as pltpu
from jax.experimental.pallas import tpu_sc as plsc

B = 16384
V = 1000000
D = 16
EPS = 1e-3
NC = 2
NS = 16
NW = NC * NS
BPW = B // NW
NK = BPW // 128
NF = 13


def _rsqrt16(x):
    i = lax.bitcast_convert_type(x, jnp.int32)
    i = jnp.int32(0x5F3759DF) - lax.shift_right_arithmetic(i, 1)
    y = lax.bitcast_convert_type(i, jnp.float32)
    for _ in range(3):
        y = y * (jnp.float32(1.5) - jnp.float32(0.5) * x * y * y)
    return y


def _sc_body(*refs):
    chans = refs[:D]
    idxs, vals, params, out = refs[D:D + 4]
    idx_v, tr_v, vals_v, params_v, gsem, osem = refs[D + 4:]

    wid = lax.axis_index("s") * NC + lax.axis_index("c")

    pltpu.sync_copy(params, params_v)
    pltpu.sync_copy(idxs.at[:, wid], idx_v)
    pltpu.sync_copy(vals.at[:, wid], vals_v)

    def fire(i, carry):
        f = i >> 2
        j = i & 3
        for c in range(D):
            pltpu.async_copy(
                chans[c].at[idx_v.at[f, j]], tr_v.at[f, j, c], gsem)
        return carry
    lax.fori_loop(0, NF * NK, fire, 0)

    pltpu.make_async_copy(
        out.at[0, 0, pl.ds(0, NF), :, pl.ds(0, NK * 128 // 8)]
        if False else out.at[pl.ds(0, NF), 0, pl.ds(0, NK), :, :],
        tr_v, gsem).wait()

    bn_scale = params_v[0, :]
    bn_shift = params_v[1, :]

    out_handles = []
    for f in range(NF):
        if f < 10:
            def bn_body(g, carry, f=f):
                j = g >> 3
                off = (g & 7) * 16
                for c in range(D):
                    sc = bn_scale[c]
                    sh = bn_shift[c]
                    t = tr_v[f, j, c, pl.ds(off, 16)]
                    tr_v[f, j, c, pl.ds(off, 16)] = t * sc + sh
                return carry
            lax.fori_loop(0, BPW // 16, bn_body, 0)
        else:
            k = f - 10
            gvec = params_v[2 + k, :]
            bvec = params_v[5 + k, :]

            def ln_body(g, carry, f=f, k=k, gvec=gvec, bvec=bvec):
                j = g >> 3
                off = (g & 7) * 16
                cvec = vals_v[k, pl.ds(g * 16, 16)]
                ts = [tr_v[f, j, c, pl.ds(off, 16)] for c in range(D)]
                s = ts[0]
                for c in range(1, D):
                    s = s + ts[c]
                mu = s * jnp.float32(1.0 / D)
                ctr = [t - mu for t in ts]
                v = ctr[0] * ctr[0]
                for c in range(1, D):
                    v = v + ctr[c] * ctr[c]
                var = v * jnp.float32(1.0 / D)
                r = _rsqrt16(cvec * cvec * var + jnp.float32(EPS))
                s2 = cvec * r
                for c in range(D):
                    tr_v[f, j, c, pl.ds(off, 16)] = (
                        ctr[c] * s2 * gvec[c] + bvec[c])
                return carry
            lax.fori_loop(0, BPW // 16, ln_body, 0)

        for g in range(2):
            out_handles.append(pltpu.async_copy(
                tr_v.at[f, :, pl.ds(g * 8, 8), :],
                out.at[f, g, pl.ds(wid * NK, NK), :, :],
                osem))

    for h in out_handles:
        h.wait()


def _tc_split_channels(tt):
    CB = 65536

    def body(x_ref, *o_refs):
        for c in range(D):
            o_refs[c][...] = x_ref[c, :]

    return pl.pallas_call(
        body,
        grid=(pl.cdiv(V, CB),),
        in_specs=[pl.BlockSpec((D, CB), lambda i: (0, i))],
        out_specs=[pl.BlockSpec((CB,), lambda i: (i,))] * D,
        out_shape=[jax.ShapeDtypeStruct((V,), jnp.float32)] * D,
    )(tt)


@jax.jit
def _sc_call(chans, idxs, vals, params):
    mesh = plsc.VectorSubcoreMesh(
        core_axis_name="c", subcore_axis_name="s",
        num_cores=NC, num_subcores=NS)
    return pl.kernel(
        _sc_body,
        out_type=jax.ShapeDtypeStruct((NF, 2, B // 128, 8, 128),
                                      jnp.float32),
        mesh=mesh,
        scratch_types=[
            pltpu.VMEM((NF, NK, 128), jnp.int32),
            pltpu.VMEM((NF, NK, D, 128), jnp.float32),
            pltpu.VMEM((3, BPW), jnp.float32),
            pltpu.VMEM((8, D), jnp.float32),
            pltpu.SemaphoreType.DMA,
            pltpu.SemaphoreType.DMA,
        ],
        compiler_params=pltpu.CompilerParams(
            needs_layout_passes=False, use_tc_tiling_on_sc=False),
    )(*chans, idxs, vals, params)


def kernel(uid, iid, utag1, utag2, utag3, utag4, itag1, itag2, itag3, itag4,
           itag4_origin_key, itag4_square_key, itag4_cube_key,
           itag4_origin_value, itag4_square_value, itag4_cube_value,
           table, bn_gamma, bn_beta, bn_mean, bn_var, ln_gamma, ln_beta):
    idxs = jnp.stack([uid, iid, utag1, utag2, utag3, utag4,
                      itag1, itag2, itag3, itag4,
                      itag4_origin_key, itag4_square_key, itag4_cube_key],
                     axis=0).reshape(NF, NW, NK, 128)
    vals = jnp.stack([itag4_origin_value, itag4_square_value,
                      itag4_cube_value], axis=0).reshape(3, NW, BPW)
    bn_scale = bn_gamma / jnp.sqrt(bn_var + EPS)
    bn_shift = bn_beta - bn_mean * bn_scale
    params = jnp.concatenate(
        [bn_scale[None, :], bn_shift[None, :], ln_gamma, ln_beta], axis=0)
    chans = _tc_split_channels(jnp.transpose(table))
    o = _sc_call(chans, idxs, vals, params)
    return jnp.transpose(o, (2, 4, 0, 1, 3)).reshape(B, NF, D)


# speedup vs baseline: 4.6653x; 1.0018x over previous
import jax
import jax.numpy as jnp
from jax import lax
from jax.experimental import pallas as pl
from jax.experimental.pallas import tpu as pltpu
from jax.experimental.pallas import tpu_sc as plsc

B = 16384
V = 1000000
D = 16
EPS = 1e-3
NC = 2
NS = 16
NW = NC * NS
BPW = B // NW
NK = BPW // 128
NF = 13


def _rsqrt16(x):
    i = lax.bitcast_convert_type(x, jnp.int32)
    i = jnp.int32(0x5F3759DF) - lax.shift_right_arithmetic(i, 1)
    y = lax.bitcast_convert_type(i, jnp.float32)
    for _ in range(3):
        y = y * (jnp.float32(1.5) - jnp.float32(0.5) * x * y * y)
    return y


def _sc_body(*refs):
    chans = refs[:D]
    idxs, vals, params, out = refs[D:D + 4]
    idx_v, tr_v, vals_v, params_v, gsem, osem = refs[D + 4:]

    wid = lax.axis_index("s") * NC + lax.axis_index("c")

    pltpu.sync_copy(params, params_v)
    pltpu.sync_copy(idxs.at[:, wid], idx_v)
    pltpu.sync_copy(vals.at[:, wid], vals_v)

    def fire(i, carry):
        f = i >> 2
        j = i & 3
        for c in range(D):
            pltpu.async_copy(
                chans[c].at[idx_v.at[f, j]], tr_v.at[f, j, c], gsem)
        return carry
    lax.fori_loop(0, NF * NK, fire, 0)

    pltpu.make_async_copy(
        out.at[pl.ds(0, NF), 0, pl.ds(0, NK), :, :], tr_v, gsem).wait()

    bn_scale = params_v[0, :]
    bn_shift = params_v[1, :]

    out_handles = []
    for f in range(NF):
        if f < 10:
            def bn_body(g, carry, f=f):
                j = g >> 3
                off = (g & 7) * 16
                for c in range(D):
                    sc = bn_scale[c]
                    sh = bn_shift[c]
                    t = tr_v[f, j, c, pl.ds(off, 16)]
                    tr_v[f, j, c, pl.ds(off, 16)] = t * sc + sh
                return carry
            lax.fori_loop(0, BPW // 16, bn_body, 0)
        else:
            k = f - 10
            gvec = params_v[2 + k, :]
            bvec = params_v[5 + k, :]

            def ln_body(g, carry, f=f, k=k, gvec=gvec, bvec=bvec):
                j = g >> 3
                off = (g & 7) * 16
                cvec = vals_v[k, pl.ds(g * 16, 16)]
                ts = [tr_v[f, j, c, pl.ds(off, 16)] for c in range(D)]
                s = ts[0]
                for c in range(1, D):
                    s = s + ts[c]
                mu = s * jnp.float32(1.0 / D)
                ctr = [t - mu for t in ts]
                v = ctr[0] * ctr[0]
                for c in range(1, D):
                    v = v + ctr[c] * ctr[c]
                var = v * jnp.float32(1.0 / D)
                r = _rsqrt16(cvec * cvec * var + jnp.float32(EPS))
                s2 = cvec * r
                for c in range(D):
                    tr_v[f, j, c, pl.ds(off, 16)] = (
                        ctr[c] * s2 * gvec[c] + bvec[c])
                return carry
            lax.fori_loop(0, BPW // 16, ln_body, 0)

        for g in range(2):
            out_handles.append(pltpu.async_copy(
                tr_v.at[f, :, pl.ds(g * 8, 8), :],
                out.at[f, g, pl.ds(wid * NK, NK), :, :],
                osem))

    for h in out_handles:
        h.wait()


def _tc_split_channels(tt):
    CB = 65536

    def body(x_ref, *o_refs):
        for c in range(D):
            o_refs[c][...] = x_ref[c, :]

    return pl.pallas_call(
        body,
        grid=(pl.cdiv(V, CB),),
        in_specs=[pl.BlockSpec((D, CB), lambda i: (0, i))],
        out_specs=[pl.BlockSpec((CB,), lambda i: (i,))] * D,
        out_shape=[jax.ShapeDtypeStruct((V,), jnp.float32)] * D,
    )(tt)


@jax.jit
def _sc_call(chans, idxs, vals, params):
    mesh = plsc.VectorSubcoreMesh(
        core_axis_name="c", subcore_axis_name="s",
        num_cores=NC, num_subcores=NS)
    return pl.kernel(
        _sc_body,
        out_type=jax.ShapeDtypeStruct((NF, 2, B // 128, 8, 128),
                                      jnp.float32),
        mesh=mesh,
        scratch_types=[
            pltpu.VMEM((NF, NK, 128), jnp.int32),
            pltpu.VMEM((NF, NK, D, 128), jnp.float32),
            pltpu.VMEM((3, BPW), jnp.float32),
            pltpu.VMEM((8, D), jnp.float32),
            pltpu.SemaphoreType.DMA,
            pltpu.SemaphoreType.DMA,
        ],
        compiler_params=pltpu.CompilerParams(
            needs_layout_passes=False, use_tc_tiling_on_sc=False),
    )(*chans, idxs, vals, params)


def kernel(uid, iid, utag1, utag2, utag3, utag4, itag1, itag2, itag3, itag4,
           itag4_origin_key, itag4_square_key, itag4_cube_key,
           itag4_origin_value, itag4_square_value, itag4_cube_value,
           table, bn_gamma, bn_beta, bn_mean, bn_var, ln_gamma, ln_beta):
    idxs = jnp.stack([uid, iid, utag1, utag2, utag3, utag4,
                      itag1, itag2, itag3, itag4,
                      itag4_origin_key, itag4_square_key, itag4_cube_key],
                     axis=0).reshape(NF, NW, NK, 128)
    vals = jnp.stack([itag4_origin_value, itag4_square_value,
                      itag4_cube_value], axis=0).reshape(3, NW, BPW)
    bn_scale = bn_gamma / jnp.sqrt(bn_var + EPS)
    bn_shift = bn_beta - bn_mean * bn_scale
    params = jnp.concatenate(
        [bn_scale[None, :], bn_shift[None, :], ln_gamma, ln_beta], axis=0)
    chans = _tc_split_channels(jnp.transpose(table))
    o = _sc_call(chans, idxs, vals, params)
    return jnp.transpose(o, (2, 4, 0, 1, 3)).reshape(B, NF, D)
